# baseline (speedup 1.0000x reference)
import jax
import jax.numpy as jnp
from jax.experimental import pallas as pl
from jax.experimental.pallas import tpu as pltpu

_N = 4096
_BLK = 256
_ATOL = 1e-8


def _check_kernel(x_ref, out_ref):
    bi = pl.program_id(0)
    i = jax.lax.broadcasted_iota(jnp.int32, (_BLK, _BLK), 0)
    j = jax.lax.broadcasted_iota(jnp.int32, (_BLK, _BLK), 1)
    mask = i >= j
    r = jnp.where(j > i, x_ref[...], 0.0)
    bad = mask & jnp.logical_not(jnp.abs(r) <= _ATOL)
    anybad = jnp.any(bad).astype(jnp.int32)

    @pl.when(bi == 0)
    def _init():
        out_ref[0, 0] = 1 - anybad

    @pl.when(bi != 0)
    def _acc():
        out_ref[0, 0] = out_ref[0, 0] * (1 - anybad)


def kernel(x):
    ok = pl.pallas_call(
        _check_kernel,
        grid=(_N // _BLK,),
        in_specs=[pl.BlockSpec((_BLK, _BLK), lambda bi: (bi, bi))],
        out_specs=pl.BlockSpec(
            (1, 1), lambda bi: (0, 0), memory_space=pltpu.SMEM
        ),
        out_shape=jax.ShapeDtypeStruct((1, 1), jnp.int32),
    )(x)
    return jnp.reshape(ok != 0, (1,))


# speedup vs baseline: 1.7598x; 1.7598x over previous
import jax
import jax.numpy as jnp
from jax.experimental import pallas as pl
from jax.experimental.pallas import tpu as pltpu

_N = 4096
_BLK = 512
_NT = _N // _BLK


def _check_kernel(*refs):
    x_refs, out_ref = refs[:-1], refs[-1]
    i = jax.lax.broadcasted_iota(jnp.int32, (_BLK, _BLK), 0)
    j = jax.lax.broadcasted_iota(jnp.int32, (_BLK, _BLK), 1)
    mask = i >= j
    bad = jnp.zeros((_BLK, _BLK), jnp.bool_)
    for x_ref in x_refs:
        r = jnp.where(j > i, x_ref[...], 0.0)
        bad |= mask & jnp.logical_not(jnp.abs(r) <= 1e-8)
    out_ref[0, 0] = jnp.logical_not(jnp.any(bad)).astype(jnp.int32)


def kernel(x):
    in_specs = [
        pl.BlockSpec((_BLK, _BLK), lambda g, k=k: (k, k))
        for k in range(_NT)
    ]
    ok = pl.pallas_call(
        _check_kernel,
        grid=(1,),
        in_specs=in_specs,
        out_specs=pl.BlockSpec(
            (1, 1), lambda g: (0, 0), memory_space=pltpu.SMEM
        ),
        out_shape=jax.ShapeDtypeStruct((1, 1), jnp.int32),
    )(*([x] * _NT))
    return jnp.reshape(ok != 0, (1,))


# speedup vs baseline: 2.1327x; 1.2119x over previous
import jax
import jax.numpy as jnp
from jax.experimental import pallas as pl
from jax.experimental.pallas import tpu as pltpu

_N = 4096
_BLK = 256
_NT = _N // _BLK


def _check_kernel(*refs):
    x_refs, out_ref = refs[:-1], refs[-1]
    i = jax.lax.broadcasted_iota(jnp.int32, (_BLK, _BLK), 0)
    j = jax.lax.broadcasted_iota(jnp.int32, (_BLK, _BLK), 1)
    mask = i >= j
    bad = jnp.zeros((_BLK, _BLK), jnp.bool_)
    for x_ref in x_refs:
        r = jnp.where(j > i, x_ref[...], 0.0)
        bad |= mask & jnp.logical_not(jnp.abs(r) <= 1e-8)
    out_ref[0, 0] = jnp.logical_not(jnp.any(bad)).astype(jnp.int32)


def kernel(x):
    in_specs = [
        pl.BlockSpec((_BLK, _BLK), lambda g, k=k: (k, k))
        for k in range(_NT)
    ]
    ok = pl.pallas_call(
        _check_kernel,
        grid=(1,),
        in_specs=in_specs,
        out_specs=pl.BlockSpec(
            (1, 1), lambda g: (0, 0), memory_space=pltpu.SMEM
        ),
        out_shape=jax.ShapeDtypeStruct((1, 1), jnp.int32),
    )(*([x] * _NT))
    return jnp.reshape(ok != 0, (1,))


# speedup vs baseline: 2.4739x; 1.1600x over previous
import jax
import jax.numpy as jnp
from jax.experimental import pallas as pl
from jax.experimental.pallas import tpu as pltpu

_N = 4096
_BLK = 128
_NT = _N // _BLK


def _check_kernel(*refs):
    x_refs, out_ref = refs[:-1], refs[-1]
    i = jax.lax.broadcasted_iota(jnp.int32, (_BLK, _BLK), 0)
    j = jax.lax.broadcasted_iota(jnp.int32, (_BLK, _BLK), 1)
    mask = i >= j
    bad = jnp.zeros((_BLK, _BLK), jnp.bool_)
    for x_ref in x_refs:
        r = jnp.where(j > i, x_ref[...], 0.0)
        bad |= mask & jnp.logical_not(jnp.abs(r) <= 1e-8)
    out_ref[0, 0] = jnp.logical_not(jnp.any(bad)).astype(jnp.int32)


def kernel(x):
    in_specs = [
        pl.BlockSpec((_BLK, _BLK), lambda g, k=k: (k, k))
        for k in range(_NT)
    ]
    ok = pl.pallas_call(
        _check_kernel,
        grid=(1,),
        in_specs=in_specs,
        out_specs=pl.BlockSpec(
            (1, 1), lambda g: (0, 0), memory_space=pltpu.SMEM
        ),
        out_shape=jax.ShapeDtypeStruct((1, 1), jnp.int32),
    )(*([x] * _NT))
    return jnp.reshape(ok != 0, (1,))


# speedup vs baseline: 2.8533x; 1.1534x over previous
import jax
import jax.numpy as jnp
from jax.experimental import pallas as pl
from jax.experimental.pallas import tpu as pltpu

_N = 4096
_BLK = 128
_NT = _N // _BLK


def _check_kernel(*refs):
    x_refs, out_ref = refs[:-1], refs[-1]
    i = jax.lax.broadcasted_iota(jnp.int32, (_BLK, _BLK), 0)
    j = jax.lax.broadcasted_iota(jnp.int32, (_BLK, _BLK), 1)
    mask = i >= j
    acc = jnp.zeros((_BLK, _BLK), jnp.float32)
    for x_ref in x_refs:
        r = jnp.where(j > i, x_ref[...], 0.0)
        m = jnp.where(mask, r, 0.0)
        acc = jnp.maximum(acc, jnp.abs(m))
    worst = jnp.max(acc)
    out_ref[0] = worst <= 1e-8


def kernel(x):
    in_specs = [
        pl.BlockSpec((_BLK, _BLK), lambda g, k=k: (k, k))
        for k in range(_NT)
    ]
    return pl.pallas_call(
        _check_kernel,
        grid=(1,),
        in_specs=in_specs,
        out_specs=pl.BlockSpec((1,), lambda g: (0,), memory_space=pltpu.SMEM),
        out_shape=jax.ShapeDtypeStruct((1,), jnp.bool_),
    )(*([x] * _NT))
